# Initial kernel scaffold; baseline (speedup 1.0000x reference)
import jax
import jax.numpy as jnp
from jax.experimental import pallas as pl


def kernel(x, values, idx_in, idx_out):
    raise NotImplementedError("write your pallas kernel here")


# speedup vs baseline: 23.5186x; 23.5186x over previous
import functools

import jax
import jax.numpy as jnp
import numpy as np
from jax.experimental import pallas as pl

_N = 4096


def _butterfly_indices(n):
    indices_in = [[0, 0, 1, 1]]
    indices_out = [[0, 1, 0, 1]]
    curr_n = 2
    while curr_n < n:
        for i in range(len(indices_in)):
            indices_in[i] = indices_in[i] + [p + curr_n for p in indices_in[i]]
            indices_out[i] = indices_out[i] + [p + curr_n for p in indices_out[i]]
        sublist_low = list(range(curr_n)) * 2
        sublist_high = list(range(curr_n, curr_n * 2)) * 2
        new_idx_in = sublist_low + sublist_high
        indices_in.append(new_idx_in)
        new_idx_out = list(range(curr_n * 2)) * 2
        indices_out.append(new_idx_out)
        indices_in.insert(0, list(new_idx_in))
        indices_out.insert(0, list(new_idx_out))
        curr_n *= 2
    return (np.array(indices_in, dtype=np.int64),
            np.array(indices_out, dtype=np.int64))


def _layer_structure(n):
    ii, io = _butterfly_indices(n)
    L, nnz = ii.shape
    strides = []
    pos_a = np.zeros((L, n), dtype=np.int32)
    pos_b = np.zeros((L, n), dtype=np.int32)
    for l in range(L):
        diag = ii[l] == io[l]
        pos_a[l, io[l, diag]] = np.nonzero(diag)[0]
        pos_b[l, io[l, ~diag]] = np.nonzero(~diag)[0]
        s = int(abs(ii[l, ~diag][0] - io[l, ~diag][0]))
        strides.append(s)
    return strides, pos_a, pos_b


_STRIDES, _POS_A, _POS_B = _layer_structure(_N)
_L = len(_STRIDES)
_FRONT = [l for l in range(_L) if _STRIDES[l] >= 128 and l < _L // 2]
_BACK = [l for l in range(_L) if _STRIDES[l] >= 128 and l > _L // 2]
_MID = [l for l in range(_L) if _STRIDES[l] < 128]


def _swap(x, s):
    n = x.shape[-1]
    parts = []
    for g in range(n // (2 * s)):
        parts.append(x[:, g * 2 * s + s:(g + 1) * 2 * s])
        parts.append(x[:, g * 2 * s:g * 2 * s + s])
    return jnp.concatenate(parts, axis=-1)


def _tile_body(x_ref, af_ref, bf_ref, ab_ref, bb_ref, m_ref, o_ref):
    x = x_ref[...]
    for j, l in enumerate(_FRONT):
        s = _STRIDES[l]
        x = af_ref[j][None, :] * x + bf_ref[j][None, :] * _swap(x, s)
    chunks = []
    for c in range(_N // 128):
        xc = x[:, c * 128:(c + 1) * 128]
        yc = jax.lax.dot_general(
            xc, m_ref[c],
            dimension_numbers=(((1,), (1,)), ((), ())),
            preferred_element_type=jnp.float32)
        chunks.append(yc)
    x = jnp.concatenate(chunks, axis=-1)
    for j, l in enumerate(_BACK):
        s = _STRIDES[l]
        x = ab_ref[j][None, :] * x + bb_ref[j][None, :] * _swap(x, s)
    o_ref[...] = x


@functools.partial(jax.jit, static_argnames=())
def kernel(x, values, idx_in, idx_out):
    del idx_in, idx_out
    batch, n = x.shape
    assert n == _N
    A = jnp.take_along_axis(values, jnp.asarray(_POS_A), axis=1)
    B = jnp.take_along_axis(values, jnp.asarray(_POS_B), axis=1)
    M = jnp.broadcast_to(jnp.eye(128, dtype=jnp.float32), (n // 128, 128, 128))
    for l in _MID:
        s = _STRIDES[l]
        a = A[l].reshape(n // 128, 128, 1)
        b = B[l].reshape(n // 128, 128, 1)
        Mp = M.reshape(n // 128, 128 // (2 * s), 2, s, 128)[:, :, ::-1]
        M = a * M + b * Mp.reshape(n // 128, 128, 128)

    af = A[jnp.asarray(_FRONT)]
    bf = B[jnp.asarray(_FRONT)]
    ab = A[jnp.asarray(_BACK)]
    bb = B[jnp.asarray(_BACK)]

    bt = 256
    if batch % bt:
        bt = batch
    nb = batch // bt
    nf = len(_FRONT)
    return pl.pallas_call(
        _tile_body,
        grid=(nb,),
        in_specs=[
            pl.BlockSpec((bt, n), lambda i: (i, 0)),
            pl.BlockSpec((nf, n), lambda i: (0, 0)),
            pl.BlockSpec((nf, n), lambda i: (0, 0)),
            pl.BlockSpec((nf, n), lambda i: (0, 0)),
            pl.BlockSpec((nf, n), lambda i: (0, 0)),
            pl.BlockSpec((n // 128, 128, 128), lambda i: (0, 0, 0)),
        ],
        out_specs=pl.BlockSpec((bt, n), lambda i: (i, 0)),
        out_shape=jax.ShapeDtypeStruct((batch, n), jnp.float32),
    )(x, af, bf, ab, bb, M)


# speedup vs baseline: 55.5760x; 2.3631x over previous
import jax
import jax.numpy as jnp
import numpy as np
from jax import lax
from jax.experimental import pallas as pl

_N = 4096


def _butterfly_strides(n):
    down = []
    s = n // 2
    while s >= 1:
        down.append(s)
        s //= 2
    return down + down[::-1][1:]


_STRIDES = _butterfly_strides(_N)
_L = len(_STRIDES)
_FRONT = [l for l in range(_L) if _STRIDES[l] >= 128 and l < _L // 2]
_BACK = [l for l in range(_L) if _STRIDES[l] >= 128 and l > _L // 2]
_MID = [l for l in range(_L) if _STRIDES[l] < 128]


def _extract_ab(values):
    a_rows, b_rows = [], []
    for l in range(_L):
        s = _STRIDES[l]
        g = _N // (2 * s)
        v = values[l].reshape(g, 2, 2, s)
        a_rows.append(jnp.stack((v[:, 0, 0], v[:, 1, 1]), axis=1).reshape(-1))
        b_rows.append(jnp.stack((v[:, 1, 0], v[:, 0, 1]), axis=1).reshape(-1))
    return jnp.stack(a_rows), jnp.stack(b_rows)


def _prep_body(amt_ref, bmt_ref, m_ref):
    row = lax.broadcasted_iota(jnp.int32, (_N, 128), 0)
    col = lax.broadcasted_iota(jnp.int32, (_N, 128), 1)
    m = jnp.where((row % 128) == col, 1.0, 0.0).astype(jnp.float32)
    for t, l in enumerate(_MID):
        s = _STRIDES[l]
        a = amt_ref[:, t:t + 1]
        b = bmt_ref[:, t:t + 1]
        take_lo = (row % (2 * s)) < s
        swapped = jnp.where(take_lo, jnp.roll(m, -s, axis=0),
                            jnp.roll(m, s, axis=0))
        m = a * m + b * swapped
    m_ref[...] = m


def _swap(x, s):
    n = x.shape[-1]
    parts = []
    for g in range(n // (2 * s)):
        parts.append(x[:, g * 2 * s + s:(g + 1) * 2 * s])
        parts.append(x[:, g * 2 * s:g * 2 * s + s])
    return jnp.concatenate(parts, axis=-1)


def _tile_body(x_ref, af_ref, bf_ref, ab_ref, bb_ref, m_ref, o_ref):
    x = x_ref[...]
    for j, l in enumerate(_FRONT):
        x = af_ref[j][None, :] * x + bf_ref[j][None, :] * _swap(x, _STRIDES[l])
    chunks = []
    for c in range(_N // 128):
        xc = x[:, c * 128:(c + 1) * 128]
        mc = m_ref[c * 128:(c + 1) * 128, :]
        chunks.append(jax.lax.dot_general(
            xc, mc,
            dimension_numbers=(((1,), (1,)), ((), ())),
            preferred_element_type=jnp.float32))
    x = jnp.concatenate(chunks, axis=-1)
    for j, l in enumerate(_BACK):
        x = ab_ref[j][None, :] * x + bb_ref[j][None, :] * _swap(x, _STRIDES[l])
    o_ref[...] = x


def kernel(x, values, idx_in, idx_out):
    del idx_in, idx_out
    batch, n = x.shape
    assert n == _N
    A, B = _extract_ab(values)

    amt = A[jnp.asarray(_MID)].T
    bmt = B[jnp.asarray(_MID)].T
    M = pl.pallas_call(
        _prep_body,
        out_shape=jax.ShapeDtypeStruct((n, 128), jnp.float32),
    )(amt, bmt)

    af = A[jnp.asarray(_FRONT)]
    bf = B[jnp.asarray(_FRONT)]
    ab = A[jnp.asarray(_BACK)]
    bb = B[jnp.asarray(_BACK)]

    bt = 256
    if batch % bt:
        bt = batch
    nb = batch // bt
    nf = len(_FRONT)
    return pl.pallas_call(
        _tile_body,
        grid=(nb,),
        in_specs=[
            pl.BlockSpec((bt, n), lambda i: (i, 0)),
            pl.BlockSpec((nf, n), lambda i: (0, 0)),
            pl.BlockSpec((nf, n), lambda i: (0, 0)),
            pl.BlockSpec((nf, n), lambda i: (0, 0)),
            pl.BlockSpec((nf, n), lambda i: (0, 0)),
            pl.BlockSpec((n, 128), lambda i: (0, 0)),
        ],
        out_specs=pl.BlockSpec((bt, n), lambda i: (i, 0)),
        out_shape=jax.ShapeDtypeStruct((batch, n), jnp.float32),
    )(x, af, bf, ab, bb, M)


# speedup vs baseline: 102.7702x; 1.8492x over previous
import jax
import jax.numpy as jnp
import numpy as np
from jax import lax
from jax.experimental import pallas as pl

_N = 4096


def _butterfly_positions(n):
    indices_in = [[0, 0, 1, 1]]
    indices_out = [[0, 1, 0, 1]]
    curr_n = 2
    while curr_n < n:
        for i in range(len(indices_in)):
            indices_in[i] = indices_in[i] + [p + curr_n for p in indices_in[i]]
            indices_out[i] = indices_out[i] + [p + curr_n for p in indices_out[i]]
        sublist_low = list(range(curr_n)) * 2
        sublist_high = list(range(curr_n, curr_n * 2)) * 2
        new_idx_in = sublist_low + sublist_high
        indices_in.append(new_idx_in)
        new_idx_out = list(range(curr_n * 2)) * 2
        indices_out.append(new_idx_out)
        indices_in.insert(0, list(new_idx_in))
        indices_out.insert(0, list(new_idx_out))
        curr_n *= 2
    ii = np.array(indices_in)
    io = np.array(indices_out)
    L = ii.shape[0]
    pos_a = np.zeros((L, n), dtype=np.int64)
    pos_b = np.zeros((L, n), dtype=np.int64)
    strides = []
    for l in range(L):
        diag = ii[l] == io[l]
        pos_a[l, io[l, diag]] = np.nonzero(diag)[0]
        pos_b[l, io[l, ~diag]] = np.nonzero(~diag)[0]
        strides.append(int(abs(ii[l, ~diag][0] - io[l, ~diag][0])))
    return strides, pos_a, pos_b


_STRIDES, _POS_A, _POS_B = _butterfly_positions(_N)
_L = len(_STRIDES)
_FRONT = [l for l in range(_L) if _STRIDES[l] >= 128 and l < _L // 2]
_BACK = [l for l in range(_L) if _STRIDES[l] >= 128 and l > _L // 2]
_MID = [l for l in range(_L) if _STRIDES[l] < 128]
_BIG = _FRONT + _BACK


def _selection_constants():
    sa = np.zeros((len(_MID), 256, 128), dtype=np.float32)
    sb = np.zeros((len(_MID), 256, 128), dtype=np.float32)
    for t, l in enumerate(_MID):
        for s_mat, pos in ((sa, _POS_A), (sb, _POS_B)):
            p = pos[l].reshape(32, 128)
            off = p - 256 * np.arange(32)[:, None]
            assert (off == off[0]).all()
            s_mat[t, off[0], np.arange(128)] = 1.0
    pa = np.zeros((len(_BIG), 32, 64), dtype=np.float32)
    pb = np.zeros((len(_BIG), 32, 64), dtype=np.float32)
    for t, l in enumerate(_BIG):
        for p_mat, pos in ((pa, _POS_A), (pb, _POS_B)):
            p = pos[l].reshape(32, 128)
            assert (p % 128 == np.arange(128)[None, :]).all()
            p_mat[t, np.arange(32), p[:, 0] // 128] = 1.0
    return sa, sb, pa, pb


_SA, _SB, _PA, _PB = _selection_constants()


def _extract_ab(values):
    vmid = values[_MID[0]:_MID[-1] + 1].reshape(len(_MID), 32, 256)
    dn = (((2,), (1,)), ((0,), (0,)))
    a_mid = lax.dot_general(vmid, jnp.asarray(_SA), dn,
                            preferred_element_type=jnp.float32)
    b_mid = lax.dot_general(vmid, jnp.asarray(_SB), dn,
                            preferred_element_type=jnp.float32)
    vbig = jnp.concatenate(
        [values[:_MID[0]], values[_MID[-1] + 1:]], axis=0).reshape(
            len(_BIG), 64, 128)
    dn2 = (((2,), (1,)), ((0,), (0,)))
    a_big = lax.dot_general(jnp.asarray(_PA), vbig, dn2,
                            preferred_element_type=jnp.float32)
    b_big = lax.dot_general(jnp.asarray(_PB), vbig, dn2,
                            preferred_element_type=jnp.float32)
    return (a_mid.reshape(len(_MID), _N), b_mid.reshape(len(_MID), _N),
            a_big.reshape(len(_BIG), _N), b_big.reshape(len(_BIG), _N))


def _prep_body(amt_ref, bmt_ref, m_ref):
    row = lax.broadcasted_iota(jnp.int32, (_N, 128), 0)
    col = lax.broadcasted_iota(jnp.int32, (_N, 128), 1)
    m = jnp.where((row % 128) == col, 1.0, 0.0).astype(jnp.float32)
    for t, l in enumerate(_MID):
        s = _STRIDES[l]
        a = amt_ref[:, t:t + 1]
        b = bmt_ref[:, t:t + 1]
        take_lo = (row & s) == 0
        swapped = jnp.where(take_lo, jnp.roll(m, -s, axis=0),
                            jnp.roll(m, s, axis=0))
        m = a * m + b * swapped
    m_ref[...] = m


def _swap(x, s):
    n = x.shape[-1]
    parts = []
    for g in range(n // (2 * s)):
        parts.append(x[:, g * 2 * s + s:(g + 1) * 2 * s])
        parts.append(x[:, g * 2 * s:g * 2 * s + s])
    return jnp.concatenate(parts, axis=-1)


def _tile_body(x_ref, af_ref, bf_ref, ab_ref, bb_ref, m_ref, o_ref):
    x = x_ref[...]
    for j, l in enumerate(_FRONT):
        x = af_ref[j][None, :] * x + bf_ref[j][None, :] * _swap(x, _STRIDES[l])
    chunks = []
    for c in range(_N // 128):
        xc = x[:, c * 128:(c + 1) * 128]
        mc = m_ref[c * 128:(c + 1) * 128, :]
        chunks.append(jax.lax.dot_general(
            xc, mc,
            dimension_numbers=(((1,), (1,)), ((), ())),
            preferred_element_type=jnp.float32))
    x = jnp.concatenate(chunks, axis=-1)
    for j, l in enumerate(_BACK):
        x = ab_ref[j][None, :] * x + bb_ref[j][None, :] * _swap(x, _STRIDES[l])
    o_ref[...] = x


def kernel(x, values, idx_in, idx_out):
    del idx_in, idx_out
    batch, n = x.shape
    assert n == _N
    a_mid, b_mid, a_big, b_big = _extract_ab(values)

    amt = a_mid.T
    bmt = b_mid.T
    M = pl.pallas_call(
        _prep_body,
        out_shape=jax.ShapeDtypeStruct((n, 128), jnp.float32),
    )(amt, bmt)

    nf = len(_FRONT)
    af, ab = a_big[:nf], a_big[nf:]
    bf, bb = b_big[:nf], b_big[nf:]

    bt = 256
    if batch % bt:
        bt = batch
    nb = batch // bt
    return pl.pallas_call(
        _tile_body,
        grid=(nb,),
        in_specs=[
            pl.BlockSpec((bt, n), lambda i: (i, 0)),
            pl.BlockSpec((nf, n), lambda i: (0, 0)),
            pl.BlockSpec((nf, n), lambda i: (0, 0)),
            pl.BlockSpec((nf, n), lambda i: (0, 0)),
            pl.BlockSpec((nf, n), lambda i: (0, 0)),
            pl.BlockSpec((n, 128), lambda i: (0, 0)),
        ],
        out_specs=pl.BlockSpec((bt, n), lambda i: (i, 0)),
        out_shape=jax.ShapeDtypeStruct((batch, n), jnp.float32),
    )(x, af, bf, ab, bb, M)
